# baseline (speedup 1.0000x reference)
import functools

import jax
import jax.numpy as jnp
from jax import lax
from jax.experimental import pallas as pl
from jax.experimental.pallas import tpu as pltpu
from jax.experimental.pallas import tpu_sc as plsc

VOCAB_ = 100000
HIDDEN_ = 128
CPAD = 4
SEQ_ = 200
BATCH_ = 4096

NC, NS, LANES = 2, 16, 16
UNROLL = 8
NACC = 4
GPC = 2


def _tc_matmul_body(e_ref, wt_ref, out_ref):
    out_ref[...] = lax.dot_general(
        wt_ref[...], e_ref[...],
        dimension_numbers=(((1,), (1,)), ((), ())),
        preferred_element_type=jnp.float32,
    )


def _project_table(embedding, wst):
    bv = 12544
    grid = pl.cdiv(VOCAB_, bv)
    return pl.pallas_call(
        _tc_matmul_body,
        grid=(grid,),
        in_specs=[
            pl.BlockSpec((bv, HIDDEN_), lambda i: (i, 0)),
            pl.BlockSpec((CPAD, HIDDEN_), lambda i: (0, 0)),
        ],
        out_specs=pl.BlockSpec((CPAD, bv), lambda i: (0, i)),
        out_shape=jax.ShapeDtypeStruct((CPAD, VOCAB_), jnp.float32),
    )(embedding, wst)


def _make_sc_body(n_workers, nslice, batch):
    b_per_w = batch // nslice
    groups = b_per_w // LANES
    chunks = groups // GPC

    def body(ewt_hbm, ids_hbm, out_hbm, tab_v, ids_v0, ids_v1, res_v,
             sem0, sem1):
        wid = lax.axis_index("s") * NC + lax.axis_index("c")
        if n_workers == NS:
            wid = lax.axis_index("s")
        cls = wid // nslice
        sl = wid % nslice

        pltpu.sync_copy(ewt_hbm.at[cls], tab_v)

        rows0 = lax.iota(jnp.int32, LANES)

        def fetch(cidx, buf, sem):
            src = ids_hbm.at[pl.ds((sl * chunks + cidx) * GPC * LANES,
                                   GPC * LANES), :]
            pltpu.make_async_copy(src, buf, sem).start()

        def drain(buf, sem):
            src = ids_hbm.at[pl.ds(0, GPC * LANES), :]
            pltpu.make_async_copy(src, buf, sem).wait()

        def accumulate(ids_v, g_local, g_abs):
            zero = jnp.zeros((LANES,), jnp.float32)
            rows = rows0 + g_local * LANES

            def seq_body(t, accs):
                new = list(accs)
                for j in range(UNROLL):
                    tt = t * UNROLL + j
                    col = jnp.full((LANES,), tt, jnp.int32)
                    iv = plsc.load_gather(ids_v, [rows, col])
                    new[j % NACC] = new[j % NACC] + plsc.load_gather(
                        tab_v, [iv])
                return tuple(new)

            accs = lax.fori_loop(0, SEQ_ // UNROLL, seq_body, (zero,) * NACC)
            res_v[pl.ds(g_abs * LANES, LANES)] = (
                (accs[0] + accs[1]) + (accs[2] + accs[3]))

        fetch(0, ids_v0, sem0)

        def chunk_pair(c2, _):
            c = 2 * c2
            drain(ids_v0, sem0)
            fetch(lax.rem(c + 1, chunks), ids_v1, sem1)
            for g in range(GPC):
                accumulate(ids_v0, g, c * GPC + g)
            drain(ids_v1, sem1)
            fetch(lax.rem(c + 2, chunks), ids_v0, sem0)
            for g in range(GPC):
                accumulate(ids_v1, g, (c + 1) * GPC + g)
            return 0

        lax.fori_loop(0, chunks // 2, chunk_pair, 0)
        drain(ids_v0, sem0)

        pltpu.sync_copy(res_v, out_hbm.at[cls, pl.ds(sl * b_per_w, b_per_w)])

    mesh = plsc.VectorSubcoreMesh(
        core_axis_name="c", subcore_axis_name="s",
        num_cores=n_workers // NS, num_subcores=NS)
    return functools.partial(
        pl.kernel,
        out_type=jax.ShapeDtypeStruct((CPAD, batch), jnp.float32),
        mesh=mesh,
        compiler_params=pltpu.CompilerParams(needs_layout_passes=False),
        scratch_types=[
            pltpu.VMEM((VOCAB_,), jnp.float32),
            pltpu.VMEM((GPC * LANES, SEQ_), jnp.int32),
            pltpu.VMEM((GPC * LANES, SEQ_), jnp.int32),
            pltpu.VMEM((b_per_w,), jnp.float32),
            pltpu.SemaphoreType.DMA,
            pltpu.SemaphoreType.DMA,
        ],
    )(body)


_sc_gather_sum = _make_sc_body(n_workers=NC * NS, nslice=NC * NS // CPAD,
                               batch=BATCH_)


def kernel(input_ids, embedding, W, b):
    ids = input_ids.astype(jnp.int32)
    wst = jnp.pad((W / SEQ_).astype(jnp.float32).T,
                  ((0, CPAD - W.shape[1]), (0, 0)))
    ewt = _project_table(embedding, wst)
    sums = _sc_gather_sum(ewt, ids)
    return sums[: W.shape[1]].T + b


# speedup vs baseline: 1.4045x; 1.4045x over previous
import functools

import jax
import jax.numpy as jnp
from jax import lax
from jax.experimental import pallas as pl
from jax.experimental.pallas import tpu as pltpu
from jax.experimental.pallas import tpu_sc as plsc

VOCAB_ = 100000
HIDDEN_ = 128
CPAD = 4
NPAIR = 2
SEQ_ = 200
BATCH_ = 4096

NC, NS, LANES = 2, 16, 16
UNROLL = 8


def _tc_body(e_ref, wt_ref, ids_ref, out_ref, ids3_ref):
    r = lax.dot_general(
        wt_ref[...], e_ref[...],
        dimension_numbers=(((1,), (1,)), ((), ())),
        preferred_element_type=jnp.float32,
    )
    u = lax.bitcast_convert_type(r.astype(jnp.bfloat16), jnp.uint16)
    u = u.astype(jnp.uint32).reshape(NPAIR, 2, r.shape[1])
    packed = u[:, 0, :] | (u[:, 1, :] << 16)
    out_ref[...] = lax.bitcast_convert_type(packed, jnp.int32)
    blk = ids_ref[...]
    g = blk.shape[0] // LANES
    ids3_ref[...] = (blk.reshape(g, LANES, SEQ_)
                     .transpose(0, 2, 1)
                     .reshape(g, SEQ_ * LANES))


def _project_pack_regroup(embedding, wst, ids):
    bv = 12544
    grid = pl.cdiv(VOCAB_, bv)
    bb = BATCH_ // grid
    return pl.pallas_call(
        _tc_body,
        grid=(grid,),
        in_specs=[
            pl.BlockSpec((bv, HIDDEN_), lambda i: (i, 0)),
            pl.BlockSpec((CPAD, HIDDEN_), lambda i: (0, 0)),
            pl.BlockSpec((bb, SEQ_), lambda i: (i, 0)),
        ],
        out_specs=[
            pl.BlockSpec((NPAIR, bv), lambda i: (0, i)),
            pl.BlockSpec((bb // LANES, SEQ_ * LANES), lambda i: (i, 0)),
        ],
        out_shape=[
            jax.ShapeDtypeStruct((NPAIR, VOCAB_), jnp.int32),
            jax.ShapeDtypeStruct((BATCH_ // LANES, SEQ_ * LANES), jnp.int32),
        ],
    )(embedding, wst, ids)


def _sc_body(ewt_hbm, ids_hbm, out_hbm, tab_v, ids_v0, ids_v1, res_v,
             sem0, sem1):
    nslice = NC * NS // NPAIR
    b_per_w = BATCH_ // nslice
    groups = b_per_w // LANES
    himask = jnp.int32(-65536)

    wid = lax.axis_index("s") * NC + lax.axis_index("c")
    pair = wid // nslice
    sl = wid % nslice
    gbase = sl * groups

    pltpu.sync_copy(ewt_hbm.at[pair], tab_v)

    def fetch(gidx, buf, sem):
        pltpu.make_async_copy(ids_hbm.at[gidx], buf, sem).start()

    def drain(buf, sem):
        pltpu.make_async_copy(ids_hbm.at[0], buf, sem).wait()

    def accumulate(ids_v, g):
        zero = jnp.zeros((LANES,), jnp.float32)

        def seq_body(t, accs):
            lo0, lo1, hi0, hi1 = accs
            for j in range(UNROLL):
                iv = ids_v[pl.ds((t * UNROLL + j) * LANES, LANES)]
                w = plsc.load_gather(tab_v, [iv])
                lo = plsc.bitcast(w << 16, jnp.float32)
                hi = plsc.bitcast(w & himask, jnp.float32)
                if j % 2 == 0:
                    lo0 = lo0 + lo
                    hi0 = hi0 + hi
                else:
                    lo1 = lo1 + lo
                    hi1 = hi1 + hi
            return lo0, lo1, hi0, hi1

        lo0, lo1, hi0, hi1 = lax.fori_loop(0, SEQ_ // UNROLL, seq_body,
                                           (zero,) * 4)
        res_v[0, pl.ds(g * LANES, LANES)] = lo0 + lo1
        res_v[1, pl.ds(g * LANES, LANES)] = hi0 + hi1

    fetch(gbase, ids_v0, sem0)

    def group_pair(g2, _):
        g = 2 * g2
        drain(ids_v0, sem0)
        fetch(gbase + lax.rem(g + 1, groups), ids_v1, sem1)
        accumulate(ids_v0, g)
        drain(ids_v1, sem1)
        fetch(gbase + lax.rem(g + 2, groups), ids_v0, sem0)
        accumulate(ids_v1, g + 1)
        return 0

    lax.fori_loop(0, groups // 2, group_pair, 0)
    drain(ids_v0, sem0)

    pltpu.sync_copy(res_v, out_hbm.at[pair, :, pl.ds(sl * b_per_w, b_per_w)])


_sc_gather_sum = functools.partial(
    pl.kernel,
    out_type=jax.ShapeDtypeStruct((NPAIR, 2, BATCH_), jnp.float32),
    mesh=plsc.VectorSubcoreMesh(core_axis_name="c", subcore_axis_name="s"),
    compiler_params=pltpu.CompilerParams(needs_layout_passes=False),
    scratch_types=[
        pltpu.VMEM((VOCAB_,), jnp.int32),
        pltpu.VMEM((SEQ_ * LANES,), jnp.int32),
        pltpu.VMEM((SEQ_ * LANES,), jnp.int32),
        pltpu.VMEM((2, BATCH_ // (NC * NS // NPAIR)), jnp.float32),
        pltpu.SemaphoreType.DMA,
        pltpu.SemaphoreType.DMA,
    ],
)(_sc_body)


def kernel(input_ids, embedding, W, b):
    ids = input_ids.astype(jnp.int32)
    wst = jnp.pad((W / SEQ_).astype(jnp.float32).T,
                  ((0, CPAD - W.shape[1]), (0, 0)))
    ewt, ids3 = _project_pack_regroup(embedding, wst, ids)
    sums = _sc_gather_sum(ewt, ids3)
    return sums.reshape(CPAD, BATCH_)[: W.shape[1]].T + b


# speedup vs baseline: 1.4823x; 1.0554x over previous
import functools

import jax
import jax.numpy as jnp
from jax import lax
from jax.experimental import pallas as pl
from jax.experimental.pallas import tpu as pltpu
from jax.experimental.pallas import tpu_sc as plsc

VOCAB_ = 100000
HIDDEN_ = 128
CPAD = 4
NPAIR = 2
SEQ_ = 200
BATCH_ = 4096

NC, NS, LANES = 2, 16, 16
UNROLL = 8


def _tc_body(e_ref, wt_ref, ids_ref, out_ref, ids3_ref):
    r = lax.dot_general(
        wt_ref[...], e_ref[...],
        dimension_numbers=(((1,), (1,)), ((), ())),
        preferred_element_type=jnp.float32,
    )
    u = lax.bitcast_convert_type(r.astype(jnp.bfloat16), jnp.uint16)
    u = u.astype(jnp.uint32).reshape(NPAIR, 2, r.shape[1])
    packed = u[:, 0, :] | (u[:, 1, :] << 16)
    out_ref[...] = lax.bitcast_convert_type(packed, jnp.int32)
    blk = ids_ref[...]
    g = blk.shape[0] // LANES
    ids3_ref[...] = (blk.reshape(g, LANES, SEQ_)
                     .transpose(0, 2, 1)
                     .reshape(g, SEQ_ * LANES))


def _project_pack_regroup(embedding, wst, ids):
    bv = 12544
    grid = pl.cdiv(VOCAB_, bv)
    bb = BATCH_ // grid
    return pl.pallas_call(
        _tc_body,
        grid=(grid,),
        in_specs=[
            pl.BlockSpec((bv, HIDDEN_), lambda i: (i, 0)),
            pl.BlockSpec((CPAD, HIDDEN_), lambda i: (0, 0)),
            pl.BlockSpec((bb, SEQ_), lambda i: (i, 0)),
        ],
        out_specs=[
            pl.BlockSpec((NPAIR, bv), lambda i: (0, i)),
            pl.BlockSpec((bb // LANES, SEQ_ * LANES), lambda i: (i, 0)),
        ],
        out_shape=[
            jax.ShapeDtypeStruct((NPAIR, VOCAB_), jnp.int32),
            jax.ShapeDtypeStruct((BATCH_ // LANES, SEQ_ * LANES), jnp.int32),
        ],
    )(embedding, wst, ids)


def _sc_body(ewt_hbm, ids_hbm, out_hbm, tab_v, ids_v0, ids_v1, res_v,
             tab_sh, sem0, sem1):
    nslice = NC * NS // NPAIR
    b_per_w = BATCH_ // nslice
    groups = b_per_w // LANES
    himask = jnp.int32(-65536)

    wid = lax.axis_index("s") * NC + lax.axis_index("c")
    pair = wid // nslice
    sl = wid % nslice
    gbase = sl * groups

    @pl.when(lax.axis_index("s") == 0)
    def _():
        pltpu.sync_copy(ewt_hbm, tab_sh)

    plsc.subcore_barrier()
    pltpu.sync_copy(tab_sh.at[pair], tab_v)

    def fetch(gidx, buf, sem):
        pltpu.make_async_copy(ids_hbm.at[gidx], buf, sem).start()

    def drain(buf, sem):
        pltpu.make_async_copy(ids_hbm.at[0], buf, sem).wait()

    def accumulate(ids_v, g):
        zero = jnp.zeros((LANES,), jnp.float32)

        def seq_body(t, accs):
            lo0, lo1, hi0, hi1 = accs
            for j in range(UNROLL):
                iv = ids_v[pl.ds((t * UNROLL + j) * LANES, LANES)]
                w = plsc.load_gather(tab_v, [iv])
                lo = plsc.bitcast(w << 16, jnp.float32)
                hi = plsc.bitcast(w & himask, jnp.float32)
                if j % 2 == 0:
                    lo0 = lo0 + lo
                    hi0 = hi0 + hi
                else:
                    lo1 = lo1 + lo
                    hi1 = hi1 + hi
            return lo0, lo1, hi0, hi1

        lo0, lo1, hi0, hi1 = lax.fori_loop(0, SEQ_ // UNROLL, seq_body,
                                           (zero,) * 4)
        res_v[0, pl.ds(g * LANES, LANES)] = lo0 + lo1
        res_v[1, pl.ds(g * LANES, LANES)] = hi0 + hi1

    fetch(gbase, ids_v0, sem0)

    def group_pair(g2, _):
        g = 2 * g2
        drain(ids_v0, sem0)
        fetch(gbase + lax.rem(g + 1, groups), ids_v1, sem1)
        accumulate(ids_v0, g)
        drain(ids_v1, sem1)
        fetch(gbase + lax.rem(g + 2, groups), ids_v0, sem0)
        accumulate(ids_v1, g + 1)
        return 0

    lax.fori_loop(0, groups // 2, group_pair, 0)
    drain(ids_v0, sem0)

    pltpu.sync_copy(res_v, out_hbm.at[pair, :, pl.ds(sl * b_per_w, b_per_w)])


_sc_gather_sum = functools.partial(
    pl.kernel,
    out_type=jax.ShapeDtypeStruct((NPAIR, 2, BATCH_), jnp.float32),
    mesh=plsc.VectorSubcoreMesh(core_axis_name="c", subcore_axis_name="s"),
    compiler_params=pltpu.CompilerParams(needs_layout_passes=False),
    scratch_types=[
        pltpu.VMEM((VOCAB_,), jnp.int32),
        pltpu.VMEM((SEQ_ * LANES,), jnp.int32),
        pltpu.VMEM((SEQ_ * LANES,), jnp.int32),
        pltpu.VMEM((2, BATCH_ // (NC * NS // NPAIR)), jnp.float32),
        pltpu.VMEM_SHARED((NPAIR, VOCAB_), jnp.int32),
        pltpu.SemaphoreType.DMA,
        pltpu.SemaphoreType.DMA,
    ],
)(_sc_body)


def kernel(input_ids, embedding, W, b):
    ids = input_ids.astype(jnp.int32)
    wst = jnp.pad((W / SEQ_).astype(jnp.float32).T,
                  ((0, CPAD - W.shape[1]), (0, 0)))
    ewt, ids3 = _project_pack_regroup(embedding, wst, ids)
    sums = _sc_gather_sum(ewt, ids3)
    return sums.reshape(CPAD, BATCH_)[: W.shape[1]].T + b


# speedup vs baseline: 1.5352x; 1.0357x over previous
import functools

import jax
import jax.numpy as jnp
from jax import lax
from jax.experimental import pallas as pl
from jax.experimental.pallas import tpu as pltpu
from jax.experimental.pallas import tpu_sc as plsc

VOCAB_ = 100000
HIDDEN_ = 128
CPAD = 4
NPAIR = 2
SEQ_ = 200
BATCH_ = 4096

NC, NS, LANES = 2, 16, 16
UNROLL = 8
GPC = 4


def _tc_body(e_ref, wt_ref, ids_ref, out_ref, ids3_ref):
    r = lax.dot_general(
        wt_ref[...], e_ref[...],
        dimension_numbers=(((1,), (1,)), ((), ())),
        preferred_element_type=jnp.float32,
    )
    u = lax.bitcast_convert_type(r.astype(jnp.bfloat16), jnp.uint16)
    u = u.astype(jnp.uint32).reshape(NPAIR, 2, r.shape[1])
    packed = u[:, 0, :] | (u[:, 1, :] << 16)
    out_ref[...] = lax.bitcast_convert_type(packed, jnp.int32)
    blk = ids_ref[...]
    g = blk.shape[0] // LANES
    ids3_ref[...] = (blk.reshape(g, LANES, SEQ_)
                     .transpose(0, 2, 1)
                     .reshape(g, SEQ_ * LANES))


def _project_pack_regroup(embedding, wst, ids):
    bv = 12544
    grid = pl.cdiv(VOCAB_, bv)
    bb = BATCH_ // grid
    return pl.pallas_call(
        _tc_body,
        grid=(grid,),
        in_specs=[
            pl.BlockSpec((bv, HIDDEN_), lambda i: (i, 0)),
            pl.BlockSpec((CPAD, HIDDEN_), lambda i: (0, 0)),
            pl.BlockSpec((bb, SEQ_), lambda i: (i, 0)),
        ],
        out_specs=[
            pl.BlockSpec((NPAIR, bv), lambda i: (0, i)),
            pl.BlockSpec((bb // LANES, SEQ_ * LANES), lambda i: (i, 0)),
        ],
        out_shape=[
            jax.ShapeDtypeStruct((NPAIR, VOCAB_), jnp.int32),
            jax.ShapeDtypeStruct((BATCH_ // LANES, SEQ_ * LANES), jnp.int32),
        ],
    )(embedding, wst, ids)


def _sc_body(ewt_hbm, ids_hbm, out_hbm, tab_v, ids_v0, ids_v1, res_v,
             sem0, sem1):
    nslice = NC * NS // NPAIR
    b_per_w = BATCH_ // nslice
    groups = b_per_w // LANES
    himask = jnp.int32(-65536)

    wid = lax.axis_index("s") * NC + lax.axis_index("c")
    pair = wid // nslice
    sl = wid % nslice
    gbase = sl * groups

    pltpu.sync_copy(ewt_hbm.at[pair], tab_v)

    chunks = groups // GPC

    def fetch(cidx, buf, sem):
        src = ids_hbm.at[pl.ds(gbase + cidx * GPC, GPC), :]
        pltpu.make_async_copy(src, buf, sem).start()

    def drain(buf, sem):
        src = ids_hbm.at[pl.ds(0, GPC), :]
        pltpu.make_async_copy(src, buf, sem).wait()

    def accumulate(ids_v, g_local, g_abs):
        zero = jnp.zeros((LANES,), jnp.float32)

        def seq_body(t, accs):
            lo0, lo1, hi0, hi1 = accs
            for j in range(UNROLL):
                iv = ids_v[g_local, pl.ds((t * UNROLL + j) * LANES, LANES)]
                w = plsc.load_gather(tab_v, [iv])
                lo = plsc.bitcast(w << 16, jnp.float32)
                hi = plsc.bitcast(w & himask, jnp.float32)
                if j % 2 == 0:
                    lo0 = lo0 + lo
                    hi0 = hi0 + hi
                else:
                    lo1 = lo1 + lo
                    hi1 = hi1 + hi
            return lo0, lo1, hi0, hi1

        lo0, lo1, hi0, hi1 = lax.fori_loop(0, SEQ_ // UNROLL, seq_body,
                                           (zero,) * 4)
        res_v[0, pl.ds(g_abs * LANES, LANES)] = lo0 + lo1
        res_v[1, pl.ds(g_abs * LANES, LANES)] = hi0 + hi1

    fetch(0, ids_v0, sem0)

    def chunk_pair(c2, _):
        c = 2 * c2
        drain(ids_v0, sem0)
        fetch(lax.rem(c + 1, chunks), ids_v1, sem1)
        for g in range(GPC):
            accumulate(ids_v0, g, c * GPC + g)
        drain(ids_v1, sem1)
        fetch(lax.rem(c + 2, chunks), ids_v0, sem0)
        for g in range(GPC):
            accumulate(ids_v1, g, (c + 1) * GPC + g)
        return 0

    lax.fori_loop(0, chunks // 2, chunk_pair, 0)
    drain(ids_v0, sem0)

    pltpu.sync_copy(res_v, out_hbm.at[pair, :, pl.ds(sl * b_per_w, b_per_w)])


_sc_gather_sum = functools.partial(
    pl.kernel,
    out_type=jax.ShapeDtypeStruct((NPAIR, 2, BATCH_), jnp.float32),
    mesh=plsc.VectorSubcoreMesh(core_axis_name="c", subcore_axis_name="s"),
    compiler_params=pltpu.CompilerParams(needs_layout_passes=False),
    scratch_types=[
        pltpu.VMEM((VOCAB_,), jnp.int32),
        pltpu.VMEM((GPC, SEQ_ * LANES), jnp.int32),
        pltpu.VMEM((GPC, SEQ_ * LANES), jnp.int32),
        pltpu.VMEM((2, BATCH_ // (NC * NS // NPAIR)), jnp.float32),
        pltpu.SemaphoreType.DMA,
        pltpu.SemaphoreType.DMA,
    ],
)(_sc_body)


def kernel(input_ids, embedding, W, b):
    ids = input_ids.astype(jnp.int32)
    wst = jnp.pad((W / SEQ_).astype(jnp.float32).T,
                  ((0, CPAD - W.shape[1]), (0, 0)))
    ewt, ids3 = _project_pack_regroup(embedding, wst, ids)
    sums = _sc_gather_sum(ewt, ids3)
    return sums.reshape(CPAD, BATCH_)[: W.shape[1]].T + b
